# baseline (speedup 1.0000x reference)
import jax
import jax.numpy as jnp
from jax import lax
from jax.experimental import pallas as pl
from jax.experimental.pallas import tpu as pltpu

_BS = 32
_NP = 8732
_NO = 16
_NC = 81


def _body(tbt_ref, tc_ref, pbc_ref, pb_ref, sc_ref,
          out_ref, cls_buf, acc):
    i = pl.program_id(0)

    @pl.when(i == 0)
    def _init():
        acc[0] = 0.0
        acc[1] = 0.0
        acc[2] = 0.0

    @pl.when(i < _BS)
    def _image():
        ox1 = tbt_ref[0, 0:1, :]
        oy1 = tbt_ref[0, 1:2, :]
        ox2 = tbt_ref[0, 2:3, :]
        oy2 = tbt_ref[0, 3:4, :]
        pcx = pbc_ref[:, 0:1]
        pcy = pbc_ref[:, 1:2]
        pw = pbc_ref[:, 2:3]
        ph = pbc_ref[:, 3:4]
        px1 = pcx - pw * 0.5
        py1 = pcy - ph * 0.5
        px2 = pcx + pw * 0.5
        py2 = pcy + ph * 0.5

        w = jnp.maximum(jnp.minimum(ox2, px2) - jnp.maximum(ox1, px1), 0.0)
        h = jnp.maximum(jnp.minimum(oy2, py2) - jnp.maximum(oy1, py1), 0.0)
        inter = w * h
        area_o = (ox2 - ox1) * (oy2 - oy1)
        area_p = (px2 - px1) * (py2 - py1)
        iou = inter / (area_o + area_p - inter)

        oidx = lax.broadcasted_iota(jnp.int32, (_NP, _NO), 1)
        pidx = lax.broadcasted_iota(jnp.int32, (_NP, _NO), 0)

        obj_max = jnp.max(iou, axis=1, keepdims=True)
        obj_idx = jnp.min(jnp.where(iou == obj_max, oidx, _NO),
                          axis=1, keepdims=True)
        p_max = jnp.max(iou, axis=0, keepdims=True)
        best_p = jnp.min(jnp.where(iou == p_max, pidx, _NP),
                         axis=0, keepdims=True)

        match = pidx == best_p
        forced = jnp.max(jnp.where(match, oidx, -1), axis=1,
                         keepdims=True)
        obj_idx = jnp.where(forced >= 0, forced, obj_idx)
        ov = jnp.where(forced >= 0, 1.0, obj_max)

        onehot = obj_idx == oidx
        labels = jnp.sum(jnp.where(onehot, tc_ref[0], 0),
                         axis=1, keepdims=True)
        labels = jnp.where(ov < 0.5, 0, labels)

        def gath(c):
            return jnp.sum(jnp.where(onehot, tbt_ref[0, c:c + 1, :], 0.0),
                           axis=1, keepdims=True)

        gx1, gy1, gx2, gy2 = gath(0), gath(1), gath(2), gath(3)
        cx = (gx1 + gx2) * 0.5
        cy = (gy1 + gy2) * 0.5
        bw = gx2 - gx1
        bh = gy2 - gy1
        gcx = (cx - pcx) / (pw * 0.1)
        gcy = (cy - pcy) / (ph * 0.1)
        gw = jnp.log(bw / pw) * 5.0
        gh = jnp.log(bh / ph) * 5.0

        posf = (labels != 0).astype(jnp.float32)
        pb = pb_ref[0]
        locsum = jnp.sum((jnp.abs(pb[:, 0:1] - gcx)
                          + jnp.abs(pb[:, 1:2] - gcy)
                          + jnp.abs(pb[:, 2:3] - gw)
                          + jnp.abs(pb[:, 3:4] - gh)) * posf)
        npos = jnp.sum(posf)

        s = sc_ref[0]
        m = jnp.max(s, axis=1, keepdims=True)
        se = jnp.sum(jnp.exp(s - m), axis=1, keepdims=True)
        lse = jnp.log(se) + m
        cidx = lax.broadcasted_iota(jnp.int32, (_NP, _NC), 1)
        s_at = jnp.sum(jnp.where(cidx == labels, s, 0.0),
                       axis=1, keepdims=True)
        cls = lse - s_at

        lane = lax.broadcasted_iota(jnp.int32, (_NP, 128), 1)
        cls_buf[...] = jnp.where(lane == i, cls, cls_buf[...])
        acc[0] += locsum
        acc[1] += npos
        acc[2] += jnp.sum(cls * posf)

    @pl.when(i == _BS)
    def _finalize():
        npos = acc[1]
        k = jnp.minimum((3.0 * npos).astype(jnp.int32), _NP)
        bits = lax.bitcast_convert_type(cls_buf[...], jnp.int32)

        def step(_, carry):
            lo, hi = carry
            mid = lo + ((hi - lo) >> 1)
            cnt = jnp.sum((bits >= mid).astype(jnp.int32), axis=0,
                          keepdims=True)
            ge = cnt >= k
            return jnp.where(ge, mid, lo), jnp.where(ge, hi, mid)

        lo0 = jnp.zeros((1, 128), jnp.int32)
        hi0 = jnp.full((1, 128), 0x7F800000, jnp.int32)
        lo, _ = lax.fori_loop(0, 31, step, (lo0, hi0))
        tval = lax.bitcast_convert_type(lo, jnp.float32)
        gt = bits > lo
        cnt_gt = jnp.sum(gt.astype(jnp.float32), axis=0, keepdims=True)
        sum_gt = jnp.sum(jnp.where(gt, cls_buf[...], 0.0), axis=0,
                         keepdims=True)
        per_img = sum_gt + (k.astype(jnp.float32) - cnt_gt) * tval
        lane = lax.broadcasted_iota(jnp.int32, (1, 128), 1)
        topk = jnp.sum(jnp.where(lane < _BS, per_img, 0.0))

        loss = acc[0] / (npos * 4.0) + (acc[2] + topk) / npos
        out_ref[...] = jnp.full((1, 1), loss, jnp.float32)


@jax.jit
def kernel(pred_boxes, pred_scores, true_boxes, true_classes, pboxes):
    f32 = jnp.float32
    tbt = jnp.transpose(true_boxes, (0, 2, 1))
    tc3 = true_classes.reshape(_BS, 1, _NO).astype(jnp.int32)

    last = _BS - 1
    out = pl.pallas_call(
        _body,
        grid=(_BS + 1,),
        in_specs=[
            pl.BlockSpec((1, 4, _NO), lambda i: (jnp.minimum(i, last), 0, 0)),
            pl.BlockSpec((1, 1, _NO), lambda i: (jnp.minimum(i, last), 0, 0)),
            pl.BlockSpec((_NP, 4), lambda i: (0, 0)),
            pl.BlockSpec((1, _NP, 4), lambda i: (jnp.minimum(i, last), 0, 0)),
            pl.BlockSpec((1, _NP, _NC), lambda i: (jnp.minimum(i, last), 0, 0)),
        ],
        out_specs=pl.BlockSpec((1, 1), lambda i: (0, 0)),
        out_shape=jax.ShapeDtypeStruct((1, 1), f32),
        scratch_shapes=[
            pltpu.VMEM((_NP, 128), f32),
            pltpu.SMEM((4,), f32),
        ],
        compiler_params=pltpu.CompilerParams(
            vmem_limit_bytes=100 * 1024 * 1024),
    )(tbt, tc3, pboxes, pred_boxes, pred_scores)
    return out[0, 0]


# speedup vs baseline: 3.6871x; 3.6871x over previous
import jax
import jax.numpy as jnp
from jax import lax
from jax.experimental import pallas as pl
from jax.experimental.pallas import tpu as pltpu

_BS = 32
_NP = 8732
_NO = 16
_NC = 81


def _match_body(tb_ref, tc_ref, pbx_ref, pbc_ref, pb_ref,
                lab_ref, stats_ref):
    tb = tb_ref[0]
    ox1 = tb[:, 0:1]
    oy1 = tb[:, 1:2]
    ox2 = tb[:, 2:3]
    oy2 = tb[:, 3:4]
    px1 = pbx_ref[0:1, :]
    py1 = pbx_ref[1:2, :]
    px2 = pbx_ref[2:3, :]
    py2 = pbx_ref[3:4, :]

    w = jnp.maximum(jnp.minimum(ox2, px2) - jnp.maximum(ox1, px1), 0.0)
    h = jnp.maximum(jnp.minimum(oy2, py2) - jnp.maximum(oy1, py1), 0.0)
    inter = w * h
    area_o = (ox2 - ox1) * (oy2 - oy1)
    area_p = (px2 - px1) * (py2 - py1)
    iou = inter / (area_o + area_p - inter)

    jidx = lax.broadcasted_iota(jnp.int32, (_NO, _NP), 0)
    pidx = lax.broadcasted_iota(jnp.int32, (_NO, _NP), 1)

    col_max = jnp.max(iou, axis=0, keepdims=True)
    obj_idx = jnp.min(jnp.where(iou == col_max, jidx, _NO),
                      axis=0, keepdims=True)
    row_max = jnp.max(iou, axis=1, keepdims=True)
    best_p = jnp.min(jnp.where(iou == row_max, pidx, _NP),
                     axis=1, keepdims=True)

    match = pidx == best_p
    forced_j = jnp.max(jnp.where(match, jidx, -1), axis=0, keepdims=True)
    obj_idx = jnp.where(forced_j >= 0, forced_j, obj_idx)
    ov = jnp.where(forced_j >= 0, 1.0, col_max)

    onehot = obj_idx == jidx
    tc_col = tc_ref[0]
    labels = jnp.sum(jnp.where(onehot, tc_col, 0), axis=0, keepdims=True)
    labels = jnp.where(ov < 0.5, 0, labels)
    lab_ref[0] = labels

    def gath(c):
        col = tb[:, c:c + 1]
        return jnp.sum(jnp.where(onehot, col, 0.0), axis=0, keepdims=True)

    gx1, gy1, gx2, gy2 = gath(0), gath(1), gath(2), gath(3)
    cx = (gx1 + gx2) * 0.5
    cy = (gy1 + gy2) * 0.5
    bw = gx2 - gx1
    bh = gy2 - gy1
    pcx = pbc_ref[0:1, :]
    pcy = pbc_ref[1:2, :]
    pw = pbc_ref[2:3, :]
    ph = pbc_ref[3:4, :]
    gcx = (cx - pcx) / (pw * 0.1)
    gcy = (cy - pcy) / (ph * 0.1)
    gw = jnp.log(bw / pw) * 5.0
    gh = jnp.log(bh / ph) * 5.0

    posf = (labels != 0).astype(jnp.float32)
    pb = pb_ref[0]
    locsum = (jnp.sum(jnp.abs(pb[0:1, :] - gcx) * posf)
              + jnp.sum(jnp.abs(pb[1:2, :] - gcy) * posf)
              + jnp.sum(jnp.abs(pb[2:3, :] - gw) * posf)
              + jnp.sum(jnp.abs(pb[3:4, :] - gh) * posf))
    npos = jnp.sum(posf)
    stats_ref[0, 0:1, :] = jnp.full((1, 128), locsum, jnp.float32)
    stats_ref[0, 1:2, :] = jnp.full((1, 128), npos, jnp.float32)


def _ce_body(sc_ref, lab_ref, cls_ref, cp_ref):
    s = sc_ref[0]
    m = jnp.max(s)
    e = jnp.exp(s - m)
    ones = jnp.ones((1, _NC), jnp.float32)
    se_t = lax.dot_general(ones, e, (((1,), (1,)), ((), ())),
                           preferred_element_type=jnp.float32)
    lse_t = jnp.log(se_t) + m

    lab = lab_ref[0]
    lab_c = jnp.transpose(lab, (1, 0))
    cidx = lax.broadcasted_iota(jnp.int32, (_NP, _NC), 1)
    sel = jnp.where(cidx == lab_c, s, 0.0)
    s_at_t = lax.dot_general(ones, sel, (((1,), (1,)), ((), ())),
                             preferred_element_type=jnp.float32)
    cls_t = lse_t - s_at_t
    cls_ref[0] = cls_t
    posf = (lab != 0).astype(jnp.float32)
    cp_ref[0, 0:1, :] = jnp.full((1, 128), jnp.sum(cls_t * posf),
                                 jnp.float32)


def _combine_body(cls_ref, stats_ref, cp_ref, out_ref):
    cls = cls_ref[...]
    stats = stats_ref[...]
    locsum = jnp.sum(stats[:, 0:1, 0:1])
    npos = jnp.sum(stats[:, 1:2, 0:1])
    clspos = jnp.sum(cp_ref[...][:, :, 0:1])

    k = jnp.minimum((3.0 * npos).astype(jnp.int32), _NP)
    bits = lax.bitcast_convert_type(cls, jnp.int32)

    def step(_, carry):
        lo, hi = carry
        mid = lo + ((hi - lo) >> 1)
        cnt = jnp.sum((bits >= mid).astype(jnp.int32), axis=2,
                      keepdims=True)
        ge = cnt >= k
        return jnp.where(ge, mid, lo), jnp.where(ge, hi, mid)

    lo0 = jnp.zeros((_BS, 1, 1), jnp.int32)
    hi0 = jnp.full((_BS, 1, 1), 0x7F800000, jnp.int32)
    lo, _ = lax.fori_loop(0, 31, step, (lo0, hi0))
    tval = lax.bitcast_convert_type(lo, jnp.float32)
    gt = bits > lo
    cnt_gt = jnp.sum(gt.astype(jnp.float32), axis=2, keepdims=True)
    sum_gt = jnp.sum(jnp.where(gt, cls, 0.0), axis=2, keepdims=True)
    topk = jnp.sum(sum_gt + (k.astype(jnp.float32) - cnt_gt) * tval)

    loss = locsum / (npos * 4.0) + (clspos + topk) / npos
    out_ref[...] = jnp.full((1, 1), loss, jnp.float32)


@jax.jit
def kernel(pred_boxes, pred_scores, true_boxes, true_classes, pboxes):
    f32 = jnp.float32
    pbc_t = pboxes.T
    pbx_t = jnp.concatenate([pbc_t[:2] - pbc_t[2:] / 2.0,
                             pbc_t[:2] + pbc_t[2:] / 2.0], axis=0)
    tc3 = true_classes.reshape(_BS, _NO, 1).astype(jnp.int32)
    pb_t = jnp.transpose(pred_boxes, (0, 2, 1))

    labels, stats = pl.pallas_call(
        _match_body,
        grid=(_BS,),
        in_specs=[
            pl.BlockSpec((1, _NO, 4), lambda i: (i, 0, 0)),
            pl.BlockSpec((1, _NO, 1), lambda i: (i, 0, 0)),
            pl.BlockSpec((4, _NP), lambda i: (0, 0)),
            pl.BlockSpec((4, _NP), lambda i: (0, 0)),
            pl.BlockSpec((1, 4, _NP), lambda i: (i, 0, 0)),
        ],
        out_specs=[
            pl.BlockSpec((1, 1, _NP), lambda i: (i, 0, 0)),
            pl.BlockSpec((1, 2, 128), lambda i: (i, 0, 0)),
        ],
        out_shape=[
            jax.ShapeDtypeStruct((_BS, 1, _NP), jnp.int32),
            jax.ShapeDtypeStruct((_BS, 2, 128), f32),
        ],
    )(true_boxes, tc3, pbx_t, pbc_t, pb_t)

    cls_all, clspos = pl.pallas_call(
        _ce_body,
        grid=(_BS,),
        in_specs=[
            pl.BlockSpec((1, _NP, _NC), lambda i: (i, 0, 0)),
            pl.BlockSpec((1, 1, _NP), lambda i: (i, 0, 0)),
        ],
        out_specs=[
            pl.BlockSpec((1, 1, _NP), lambda i: (i, 0, 0)),
            pl.BlockSpec((1, 1, 128), lambda i: (i, 0, 0)),
        ],
        out_shape=[
            jax.ShapeDtypeStruct((_BS, 1, _NP), f32),
            jax.ShapeDtypeStruct((_BS, 1, 128), f32),
        ],
        compiler_params=pltpu.CompilerParams(
            vmem_limit_bytes=100 * 1024 * 1024),
    )(pred_scores, labels)

    out = pl.pallas_call(
        _combine_body,
        out_shape=jax.ShapeDtypeStruct((1, 1), f32),
    )(cls_all, stats, clspos)
    return out[0, 0]


# speedup vs baseline: 4.1000x; 1.1120x over previous
import jax
import jax.numpy as jnp
from jax import lax
from jax.experimental import pallas as pl
from jax.experimental.pallas import tpu as pltpu

_BS = 32
_NP = 8732
_NO = 16
_NC = 81


def _image_body(tb_ref, tc_ref, pbx_ref, pbc_ref, pb_ref, sc_ref,
                cls_ref, stats_ref):
    tb = tb_ref[0]
    ox1 = tb[:, 0:1]
    oy1 = tb[:, 1:2]
    ox2 = tb[:, 2:3]
    oy2 = tb[:, 3:4]
    px1 = pbx_ref[0:1, :]
    py1 = pbx_ref[1:2, :]
    px2 = pbx_ref[2:3, :]
    py2 = pbx_ref[3:4, :]

    w = jnp.maximum(jnp.minimum(ox2, px2) - jnp.maximum(ox1, px1), 0.0)
    h = jnp.maximum(jnp.minimum(oy2, py2) - jnp.maximum(oy1, py1), 0.0)
    inter = w * h
    area_o = (ox2 - ox1) * (oy2 - oy1)
    area_p = (px2 - px1) * (py2 - py1)
    iou = inter / (area_o + area_p - inter)

    jidx = lax.broadcasted_iota(jnp.int32, (_NO, _NP), 0)
    pidx = lax.broadcasted_iota(jnp.int32, (_NO, _NP), 1)

    col_max = jnp.max(iou, axis=0, keepdims=True)
    obj_idx = jnp.min(jnp.where(iou == col_max, jidx, _NO),
                      axis=0, keepdims=True)
    row_max = jnp.max(iou, axis=1, keepdims=True)
    best_p = jnp.min(jnp.where(iou == row_max, pidx, _NP),
                     axis=1, keepdims=True)

    match = pidx == best_p
    forced_j = jnp.max(jnp.where(match, jidx, -1), axis=0, keepdims=True)
    obj_idx = jnp.where(forced_j >= 0, forced_j, obj_idx)
    ov = jnp.where(forced_j >= 0, 1.0, col_max)

    onehotf = (obj_idx == jidx).astype(jnp.float32)
    tb5 = jnp.concatenate([tb, tc_ref[0].astype(jnp.float32)], axis=1)
    gath = lax.dot_general(tb5, onehotf, (((0,), (0,)), ((), ())),
                           preferred_element_type=jnp.float32)
    gx1 = gath[0:1, :]
    gy1 = gath[1:2, :]
    gx2 = gath[2:3, :]
    gy2 = gath[3:4, :]
    labels = jnp.where(ov < 0.5, 0, gath[4:5, :].astype(jnp.int32))

    cx = (gx1 + gx2) * 0.5
    cy = (gy1 + gy2) * 0.5
    bw = gx2 - gx1
    bh = gy2 - gy1
    pcx = pbc_ref[0:1, :]
    pcy = pbc_ref[1:2, :]
    pw = pbc_ref[2:3, :]
    ph = pbc_ref[3:4, :]
    gcx = (cx - pcx) / (pw * 0.1)
    gcy = (cy - pcy) / (ph * 0.1)
    gw = jnp.log(bw / pw) * 5.0
    gh = jnp.log(bh / ph) * 5.0

    posf = (labels != 0).astype(jnp.float32)
    pb = pb_ref[0]
    locsum = (jnp.sum(jnp.abs(pb[0:1, :] - gcx) * posf)
              + jnp.sum(jnp.abs(pb[1:2, :] - gcy) * posf)
              + jnp.sum(jnp.abs(pb[2:3, :] - gw) * posf)
              + jnp.sum(jnp.abs(pb[3:4, :] - gh) * posf))
    npos = jnp.sum(posf)

    s = sc_ref[0]
    m = jnp.max(s)
    e = jnp.exp(s - m)
    ones = jnp.ones((1, _NC), jnp.float32)
    se_t = lax.dot_general(ones, e, (((1,), (1,)), ((), ())),
                           preferred_element_type=jnp.float32)
    lse_t = jnp.log(se_t) + m

    lab_c = jnp.transpose(labels, (1, 0))
    cidx = lax.broadcasted_iota(jnp.int32, (_NP, _NC), 1)
    sel = jnp.where(cidx == lab_c, s, 0.0)
    s_at_t = lax.dot_general(ones, sel, (((1,), (1,)), ((), ())),
                             preferred_element_type=jnp.float32)
    cls_t = lse_t - s_at_t
    cls_ref[0] = cls_t

    stats_ref[0, 0:1, :] = jnp.full((1, 128), locsum, jnp.float32)
    stats_ref[0, 1:2, :] = jnp.full((1, 128), npos, jnp.float32)
    stats_ref[0, 2:3, :] = jnp.full((1, 128), jnp.sum(cls_t * posf),
                                    jnp.float32)
    stats_ref[0, 3:4, :] = jnp.zeros((1, 128), jnp.float32)


def _combine_body(cls_ref, stats_ref, out_ref):
    cls = cls_ref[...]
    stats = stats_ref[...]
    locsum = jnp.sum(stats[:, 0:1, 0:1])
    npos = jnp.sum(stats[:, 1:2, 0:1])
    clspos = jnp.sum(stats[:, 2:3, 0:1])

    k = jnp.minimum((3.0 * npos).astype(jnp.int32), _NP)
    bits = lax.bitcast_convert_type(cls, jnp.int32)

    def step(_, carry):
        lo, hi = carry
        mid = lo + ((hi - lo) >> 1)
        cnt = jnp.sum((bits >= mid).astype(jnp.int32), axis=2,
                      keepdims=True)
        ge = cnt >= k
        return jnp.where(ge, mid, lo), jnp.where(ge, hi, mid)

    lo0 = jnp.zeros((_BS, 1, 1), jnp.int32)
    hi0 = jnp.full((_BS, 1, 1), 0x7F800000, jnp.int32)
    lo, _ = lax.fori_loop(0, 31, step, (lo0, hi0))
    tval = lax.bitcast_convert_type(lo, jnp.float32)
    gt = bits > lo
    cnt_gt = jnp.sum(gt.astype(jnp.float32), axis=2, keepdims=True)
    sum_gt = jnp.sum(jnp.where(gt, cls, 0.0), axis=2, keepdims=True)
    topk = jnp.sum(sum_gt + (k.astype(jnp.float32) - cnt_gt) * tval)

    loss = locsum / (npos * 4.0) + (clspos + topk) / npos
    out_ref[...] = jnp.full((1, 1), loss, jnp.float32)


@jax.jit
def kernel(pred_boxes, pred_scores, true_boxes, true_classes, pboxes):
    f32 = jnp.float32
    pbc_t = pboxes.T
    pbx_t = jnp.concatenate([pbc_t[:2] - pbc_t[2:] / 2.0,
                             pbc_t[:2] + pbc_t[2:] / 2.0], axis=0)
    tc3 = true_classes.reshape(_BS, _NO, 1).astype(jnp.int32)
    pb_t = jnp.transpose(pred_boxes, (0, 2, 1))

    cls_all, stats = pl.pallas_call(
        _image_body,
        grid=(_BS,),
        in_specs=[
            pl.BlockSpec((1, _NO, 4), lambda i: (i, 0, 0)),
            pl.BlockSpec((1, _NO, 1), lambda i: (i, 0, 0)),
            pl.BlockSpec((4, _NP), lambda i: (0, 0)),
            pl.BlockSpec((4, _NP), lambda i: (0, 0)),
            pl.BlockSpec((1, 4, _NP), lambda i: (i, 0, 0)),
            pl.BlockSpec((1, _NP, _NC), lambda i: (i, 0, 0)),
        ],
        out_specs=[
            pl.BlockSpec((1, 1, _NP), lambda i: (i, 0, 0)),
            pl.BlockSpec((1, 4, 128), lambda i: (i, 0, 0)),
        ],
        out_shape=[
            jax.ShapeDtypeStruct((_BS, 1, _NP), f32),
            jax.ShapeDtypeStruct((_BS, 4, 128), f32),
        ],
        compiler_params=pltpu.CompilerParams(
            vmem_limit_bytes=100 * 1024 * 1024),
    )(true_boxes, tc3, pbx_t, pbc_t, pb_t, pred_scores)

    out = pl.pallas_call(
        _combine_body,
        out_shape=jax.ShapeDtypeStruct((1, 1), f32),
    )(cls_all, stats)
    return out[0, 0]


# speedup vs baseline: 4.1182x; 1.0044x over previous
import functools

import jax
import jax.numpy as jnp
from jax import lax
from jax.experimental import pallas as pl
from jax.experimental.pallas import tpu as pltpu
from jax.experimental.pallas import tpu_sc as plsc

_BS = 32
_NP = 8732
_NPP = 8736
_NCH = _NPP // 16
_NO = 16
_NC = 81


def _sc_match_body(obj_hbm, pxy_hbm, out_hbm,
                   objv, px1v, py1v, px2v, py2v, ovb, ojb, selb):
    i = lax.axis_index("s") * 2 + lax.axis_index("c")
    pltpu.sync_copy(obj_hbm.at[i], objv)
    pltpu.sync_copy(pxy_hbm.at[0], px1v)
    pltpu.sync_copy(pxy_hbm.at[1], py1v)
    pltpu.sync_copy(pxy_hbm.at[2], px2v)
    pltpu.sync_copy(pxy_hbm.at[3], py2v)

    lane = lax.broadcasted_iota(jnp.int32, (16,), 0)
    ox1v = objv[0]
    oy1v = objv[1]
    ox2v = objv[2]
    oy2v = objv[3]
    ox1s = [ox1v[j] for j in range(_NO)]
    oy1s = [oy1v[j] for j in range(_NO)]
    ox2s = [ox2v[j] for j in range(_NO)]
    oy2s = [oy2v[j] for j in range(_NO)]

    def chunk(c, carry):
        mxs, ids = carry
        base = c * 16
        p1 = px1v[pl.ds(base, 16)]
        q1 = py1v[pl.ds(base, 16)]
        p2 = px2v[pl.ds(base, 16)]
        q2 = py2v[pl.ds(base, 16)]
        area_p = (p2 - p1) * (q2 - q1)
        bov = jnp.full((16,), -1.0, jnp.float32)
        boj = jnp.zeros((16,), jnp.int32)
        pb_idx = base + lane
        new_mxs = []
        new_ids = []
        for j in range(_NO):
            ox1 = ox1s[j]
            oy1 = oy1s[j]
            ox2 = ox2s[j]
            oy2 = oy2s[j]
            wdt = jnp.maximum(jnp.minimum(p2, ox2) - jnp.maximum(p1, ox1),
                              0.0)
            hgt = jnp.maximum(jnp.minimum(q2, oy2) - jnp.maximum(q1, oy1),
                              0.0)
            inter = wdt * hgt
            area_o = (ox2 - ox1) * (oy2 - oy1)
            iou = inter / (area_o + area_p - inter)
            upd = iou > bov
            boj = jnp.where(upd, j, boj)
            bov = jnp.where(upd, iou, bov)
            upd2 = iou > mxs[j]
            new_ids.append(jnp.where(upd2, pb_idx, ids[j]))
            new_mxs.append(jnp.where(upd2, iou, mxs[j]))
        ovb[pl.ds(base, 16)] = bov
        ojb[pl.ds(base, 16)] = boj
        return tuple(new_mxs), tuple(new_ids)

    mx0 = tuple(jnp.full((16,), -1.0, jnp.float32) for _ in range(_NO))
    id0 = tuple(jnp.zeros((16,), jnp.int32) for _ in range(_NO))
    mxs, ids = lax.fori_loop(0, _NCH, chunk, (mx0, id0))

    dn = lax.GatherDimensionNumbers(offset_dims=(),
                                    collapsed_slice_dims=(0,),
                                    start_index_map=(0,))

    def perm(x, pm):
        return lax.gather(x, pm[:, None], dn, (1,),
                          mode=lax.GatherScatterMode.PROMISE_IN_BOUNDS)

    mxs_l = list(mxs)
    ids_l = list(ids)
    for r in (8, 4, 2, 1):
        pm = (lane + r) & 15
        for j in range(_NO):
            my = perm(mxs_l[j], pm)
            iy = perm(ids_l[j], pm)
            take = (my > mxs_l[j]) | ((my == mxs_l[j]) & (iy < ids_l[j]))
            ids_l[j] = jnp.where(take, iy, ids_l[j])
            mxs_l[j] = jnp.maximum(mxs_l[j], my)

    def selchunk(c, carry):
        base = c * 16
        ovv = ovb[pl.ds(base, 16)]
        ojv = ojb[pl.ds(base, 16)]
        sel = ojv + jnp.where(ovv < 0.5, 16, 0)
        pb_idx = base + lane
        for j in range(_NO):
            sel = jnp.where(pb_idx == ids_l[j], j, sel)
        selb[pl.ds(base, 16)] = sel
        return carry

    lax.fori_loop(0, _NCH, selchunk, 0)
    pltpu.sync_copy(selb, out_hbm.at[i])


def _sc_match(obj4, pxy):
    f = pl.kernel(
        _sc_match_body,
        mesh=plsc.VectorSubcoreMesh(core_axis_name="c", subcore_axis_name="s"),
        out_type=jax.ShapeDtypeStruct((_BS, _NPP), jnp.int32),
        scratch_types=[
            pltpu.VMEM((4, 16), jnp.float32),
            pltpu.VMEM((_NPP,), jnp.float32),
            pltpu.VMEM((_NPP,), jnp.float32),
            pltpu.VMEM((_NPP,), jnp.float32),
            pltpu.VMEM((_NPP,), jnp.float32),
            pltpu.VMEM((_NPP,), jnp.float32),
            pltpu.VMEM((_NPP,), jnp.int32),
            pltpu.VMEM((_NPP,), jnp.int32),
        ],
    )
    return f(obj4, pxy)


def _image_body(sel_ref, tb_ref, tc_ref, pbc_ref, pb_ref, sc_ref,
                cls_ref, stats_ref):
    selp = sel_ref[0][:, :_NP]
    neg = selp >= 16
    obj = jnp.where(neg, selp - 16, selp)

    jidx = lax.broadcasted_iota(jnp.int32, (_NO, _NP), 0)
    onehotf = (obj == jidx).astype(jnp.float32)
    tb = tb_ref[0]
    tb5 = jnp.concatenate([tb, tc_ref[0].astype(jnp.float32)], axis=1)
    gath = lax.dot_general(tb5, onehotf, (((0,), (0,)), ((), ())),
                           preferred_element_type=jnp.float32)
    gx1 = gath[0:1, :]
    gy1 = gath[1:2, :]
    gx2 = gath[2:3, :]
    gy2 = gath[3:4, :]
    labels = jnp.where(neg, 0, gath[4:5, :].astype(jnp.int32))

    cx = (gx1 + gx2) * 0.5
    cy = (gy1 + gy2) * 0.5
    bw = gx2 - gx1
    bh = gy2 - gy1
    pcx = pbc_ref[0:1, :]
    pcy = pbc_ref[1:2, :]
    pw = pbc_ref[2:3, :]
    ph = pbc_ref[3:4, :]
    gcx = (cx - pcx) / (pw * 0.1)
    gcy = (cy - pcy) / (ph * 0.1)
    gw = jnp.log(bw / pw) * 5.0
    gh = jnp.log(bh / ph) * 5.0

    posf = (labels != 0).astype(jnp.float32)
    pb = pb_ref[0]
    locsum = (jnp.sum(jnp.abs(pb[0:1, :] - gcx) * posf)
              + jnp.sum(jnp.abs(pb[1:2, :] - gcy) * posf)
              + jnp.sum(jnp.abs(pb[2:3, :] - gw) * posf)
              + jnp.sum(jnp.abs(pb[3:4, :] - gh) * posf))
    npos = jnp.sum(posf)

    s = sc_ref[0]
    m = jnp.max(s)
    e = jnp.exp(s - m)
    ones = jnp.ones((1, _NC), jnp.float32)
    se_t = lax.dot_general(ones, e, (((1,), (1,)), ((), ())),
                           preferred_element_type=jnp.float32)
    lse_t = jnp.log(se_t) + m

    lab_c = jnp.transpose(labels, (1, 0))
    cidx = lax.broadcasted_iota(jnp.int32, (_NP, _NC), 1)
    sel = jnp.where(cidx == lab_c, s, 0.0)
    s_at_t = lax.dot_general(ones, sel, (((1,), (1,)), ((), ())),
                             preferred_element_type=jnp.float32)
    cls_t = lse_t - s_at_t
    cls_ref[0] = cls_t

    stats_ref[0, 0:1, :] = jnp.full((1, 128), locsum, jnp.float32)
    stats_ref[0, 1:2, :] = jnp.full((1, 128), npos, jnp.float32)
    stats_ref[0, 2:3, :] = jnp.full((1, 128), jnp.sum(cls_t * posf),
                                    jnp.float32)
    stats_ref[0, 3:4, :] = jnp.zeros((1, 128), jnp.float32)


def _combine_body(cls_ref, stats_ref, out_ref):
    cls = cls_ref[...]
    stats = stats_ref[...]
    locsum = jnp.sum(stats[:, 0:1, 0:1])
    npos = jnp.sum(stats[:, 1:2, 0:1])
    clspos = jnp.sum(stats[:, 2:3, 0:1])

    k = jnp.minimum((3.0 * npos).astype(jnp.int32), _NP)
    bits = lax.bitcast_convert_type(cls, jnp.int32)

    def step(_, carry):
        lo, hi = carry
        mid = lo + ((hi - lo) >> 1)
        cnt = jnp.sum((bits >= mid).astype(jnp.int32), axis=2,
                      keepdims=True)
        ge = cnt >= k
        return jnp.where(ge, mid, lo), jnp.where(ge, hi, mid)

    lo0 = jnp.zeros((_BS, 1, 1), jnp.int32)
    hi0 = jnp.full((_BS, 1, 1), 0x7F800000, jnp.int32)
    lo, _ = lax.fori_loop(0, 31, step, (lo0, hi0))
    tval = lax.bitcast_convert_type(lo, jnp.float32)
    gt = bits > lo
    cnt_gt = jnp.sum(gt.astype(jnp.float32), axis=2, keepdims=True)
    sum_gt = jnp.sum(jnp.where(gt, cls, 0.0), axis=2, keepdims=True)
    topk = jnp.sum(sum_gt + (k.astype(jnp.float32) - cnt_gt) * tval)

    loss = locsum / (npos * 4.0) + (clspos + topk) / npos
    out_ref[...] = jnp.full((1, 1), loss, jnp.float32)


@jax.jit
def kernel(pred_boxes, pred_scores, true_boxes, true_classes, pboxes):
    f32 = jnp.float32
    pbc_t = pboxes.T
    pbx_t = jnp.concatenate([pbc_t[:2] - pbc_t[2:] / 2.0,
                             pbc_t[:2] + pbc_t[2:] / 2.0], axis=0)
    pad = jnp.full((4, _NPP - _NP), 2.0, f32)
    pxy = jnp.concatenate([pbx_t, pad], axis=1)
    obj4 = jnp.transpose(true_boxes, (0, 2, 1))
    tc3 = true_classes.reshape(_BS, _NO, 1).astype(jnp.int32)
    pb_t = jnp.transpose(pred_boxes, (0, 2, 1))

    sel = _sc_match(obj4, pxy)
    sel3 = sel.reshape(_BS, 1, _NPP)

    cls_all, stats = pl.pallas_call(
        _image_body,
        grid=(_BS,),
        in_specs=[
            pl.BlockSpec((1, 1, _NPP), lambda i: (i, 0, 0)),
            pl.BlockSpec((1, _NO, 4), lambda i: (i, 0, 0)),
            pl.BlockSpec((1, _NO, 1), lambda i: (i, 0, 0)),
            pl.BlockSpec((4, _NP), lambda i: (0, 0)),
            pl.BlockSpec((1, 4, _NP), lambda i: (i, 0, 0)),
            pl.BlockSpec((1, _NP, _NC), lambda i: (i, 0, 0)),
        ],
        out_specs=[
            pl.BlockSpec((1, 1, _NP), lambda i: (i, 0, 0)),
            pl.BlockSpec((1, 4, 128), lambda i: (i, 0, 0)),
        ],
        out_shape=[
            jax.ShapeDtypeStruct((_BS, 1, _NP), f32),
            jax.ShapeDtypeStruct((_BS, 4, 128), f32),
        ],
        compiler_params=pltpu.CompilerParams(
            vmem_limit_bytes=100 * 1024 * 1024),
    )(sel3, true_boxes, tc3, pbc_t, pb_t, pred_scores)

    out = pl.pallas_call(
        _combine_body,
        out_shape=jax.ShapeDtypeStruct((1, 1), f32),
    )(cls_all, stats)
    return out[0, 0]


# speedup vs baseline: 4.4759x; 1.0868x over previous
import functools

import jax
import jax.numpy as jnp
from jax import lax
from jax.experimental import pallas as pl
from jax.experimental.pallas import tpu as pltpu
from jax.experimental.pallas import tpu_sc as plsc

_BS = 32
_NP = 8732
_NPP = 8736
_NCH = _NPP // 16
_NO = 16
_NC = 81


def _sc_match_body(obj_hbm, pxy_hbm, out_hbm,
                   objv, px1v, py1v, px2v, py2v, ovb, ojb, selb):
    i = lax.axis_index("s") * 2 + lax.axis_index("c")
    pltpu.sync_copy(obj_hbm.at[i], objv)
    pltpu.sync_copy(pxy_hbm.at[0], px1v)
    pltpu.sync_copy(pxy_hbm.at[1], py1v)
    pltpu.sync_copy(pxy_hbm.at[2], px2v)
    pltpu.sync_copy(pxy_hbm.at[3], py2v)

    lane = lax.broadcasted_iota(jnp.int32, (16,), 0)
    ox1v = objv[0]
    oy1v = objv[1]
    ox2v = objv[2]
    oy2v = objv[3]
    ox1s = [ox1v[j] for j in range(_NO)]
    oy1s = [oy1v[j] for j in range(_NO)]
    ox2s = [ox2v[j] for j in range(_NO)]
    oy2s = [oy2v[j] for j in range(_NO)]

    def chunk(c, carry):
        mxs, ids = carry
        base = c * 16
        p1 = px1v[pl.ds(base, 16)]
        q1 = py1v[pl.ds(base, 16)]
        p2 = px2v[pl.ds(base, 16)]
        q2 = py2v[pl.ds(base, 16)]
        area_p = (p2 - p1) * (q2 - q1)
        bov = jnp.full((16,), -1.0, jnp.float32)
        boj = jnp.zeros((16,), jnp.int32)
        pb_idx = base + lane
        new_mxs = []
        new_ids = []
        for j in range(_NO):
            ox1 = ox1s[j]
            oy1 = oy1s[j]
            ox2 = ox2s[j]
            oy2 = oy2s[j]
            wdt = jnp.maximum(jnp.minimum(p2, ox2) - jnp.maximum(p1, ox1),
                              0.0)
            hgt = jnp.maximum(jnp.minimum(q2, oy2) - jnp.maximum(q1, oy1),
                              0.0)
            inter = wdt * hgt
            area_o = (ox2 - ox1) * (oy2 - oy1)
            iou = inter / (area_o + area_p - inter)
            upd = iou > bov
            boj = jnp.where(upd, j, boj)
            bov = jnp.where(upd, iou, bov)
            upd2 = iou > mxs[j]
            new_ids.append(jnp.where(upd2, pb_idx, ids[j]))
            new_mxs.append(jnp.where(upd2, iou, mxs[j]))
        ovb[pl.ds(base, 16)] = bov
        ojb[pl.ds(base, 16)] = boj
        return tuple(new_mxs), tuple(new_ids)

    mx0 = tuple(jnp.full((16,), -1.0, jnp.float32) for _ in range(_NO))
    id0 = tuple(jnp.zeros((16,), jnp.int32) for _ in range(_NO))
    mxs, ids = lax.fori_loop(0, _NCH, chunk, (mx0, id0))

    dn = lax.GatherDimensionNumbers(offset_dims=(),
                                    collapsed_slice_dims=(0,),
                                    start_index_map=(0,))

    def perm(x, pm):
        return lax.gather(x, pm[:, None], dn, (1,),
                          mode=lax.GatherScatterMode.PROMISE_IN_BOUNDS)

    mxs_l = list(mxs)
    ids_l = list(ids)
    for r in (8, 4, 2, 1):
        pm = (lane + r) & 15
        for j in range(_NO):
            my = perm(mxs_l[j], pm)
            iy = perm(ids_l[j], pm)
            take = (my > mxs_l[j]) | ((my == mxs_l[j]) & (iy < ids_l[j]))
            ids_l[j] = jnp.where(take, iy, ids_l[j])
            mxs_l[j] = jnp.maximum(mxs_l[j], my)

    def selchunk(c, carry):
        base = c * 16
        ovv = ovb[pl.ds(base, 16)]
        ojv = ojb[pl.ds(base, 16)]
        sel = ojv + jnp.where(ovv < 0.5, 16, 0)
        pb_idx = base + lane
        for j in range(_NO):
            sel = jnp.where(pb_idx == ids_l[j], j, sel)
        selb[pl.ds(base, 16)] = sel
        return carry

    lax.fori_loop(0, _NCH, selchunk, 0)
    pltpu.sync_copy(selb, out_hbm.at[i])


def _sc_match(obj4, pxy):
    f = pl.kernel(
        _sc_match_body,
        mesh=plsc.VectorSubcoreMesh(core_axis_name="c", subcore_axis_name="s"),
        out_type=jax.ShapeDtypeStruct((_BS, _NPP), jnp.int32),
        scratch_types=[
            pltpu.VMEM((4, 16), jnp.float32),
            pltpu.VMEM((_NPP,), jnp.float32),
            pltpu.VMEM((_NPP,), jnp.float32),
            pltpu.VMEM((_NPP,), jnp.float32),
            pltpu.VMEM((_NPP,), jnp.float32),
            pltpu.VMEM((_NPP,), jnp.float32),
            pltpu.VMEM((_NPP,), jnp.int32),
            pltpu.VMEM((_NPP,), jnp.int32),
        ],
    )
    return f(obj4, pxy)


def _image_body(sel_ref, tb_ref, tc_ref, pbc_ref, pb_ref, sc_ref,
                cls_ref, stats_ref):
    selp = sel_ref[0][:, :_NP]
    neg = selp >= 16
    obj = jnp.where(neg, selp - 16, selp)

    jidx = lax.broadcasted_iota(jnp.int32, (_NO, _NP), 0)
    onehotf = (obj == jidx).astype(jnp.float32)
    tb = tb_ref[0]
    tb5 = jnp.concatenate([tb, tc_ref[0].astype(jnp.float32)], axis=1)
    gath = lax.dot_general(tb5, onehotf, (((0,), (0,)), ((), ())),
                           preferred_element_type=jnp.float32)
    gx1 = gath[0:1, :]
    gy1 = gath[1:2, :]
    gx2 = gath[2:3, :]
    gy2 = gath[3:4, :]
    labels = jnp.where(neg, 0, gath[4:5, :].astype(jnp.int32))

    cx = (gx1 + gx2) * 0.5
    cy = (gy1 + gy2) * 0.5
    bw = gx2 - gx1
    bh = gy2 - gy1
    pcx = pbc_ref[0:1, :]
    pcy = pbc_ref[1:2, :]
    pw = pbc_ref[2:3, :]
    ph = pbc_ref[3:4, :]
    gcx = (cx - pcx) / (pw * 0.1)
    gcy = (cy - pcy) / (ph * 0.1)
    gw = jnp.log(bw / pw) * 5.0
    gh = jnp.log(bh / ph) * 5.0

    posf = (labels != 0).astype(jnp.float32)
    pb = pb_ref[0]
    locsum = (jnp.sum(jnp.abs(pb[0:1, :] - gcx) * posf)
              + jnp.sum(jnp.abs(pb[1:2, :] - gcy) * posf)
              + jnp.sum(jnp.abs(pb[2:3, :] - gw) * posf)
              + jnp.sum(jnp.abs(pb[3:4, :] - gh) * posf))
    npos = jnp.sum(posf)

    s = sc_ref[0]
    m = jnp.max(s)
    e = jnp.exp(s - m)
    ones = jnp.ones((1, _NC), jnp.float32)
    se_t = lax.dot_general(ones, e, (((1,), (1,)), ((), ())),
                           preferred_element_type=jnp.float32)
    lse_t = jnp.log(se_t) + m

    tcc = jnp.concatenate([tc_ref[0], jnp.zeros((1, 1), jnp.int32)],
                          axis=0)
    cid81 = lax.broadcasted_iota(jnp.int32, (_NO + 1, _NC), 1)
    tco = (cid81 == tcc).astype(jnp.float32)
    cand = lax.dot_general(tco, s, (((1,), (1,)), ((), ())),
                           preferred_element_type=jnp.float32)
    k = jnp.where(neg, _NO, obj)
    rid = lax.broadcasted_iota(jnp.int32, (_NO + 1, _NP), 0)
    csel = jnp.where(rid == k, cand, 0.0)
    ones17 = jnp.ones((1, _NO + 1), jnp.float32)
    s_at_t = lax.dot_general(ones17, csel, (((1,), (0,)), ((), ())),
                             preferred_element_type=jnp.float32)
    cls_t = lse_t - s_at_t
    cls_ref[0] = cls_t

    stats_ref[0, 0:1, :] = jnp.full((1, 128), locsum, jnp.float32)
    stats_ref[0, 1:2, :] = jnp.full((1, 128), npos, jnp.float32)
    stats_ref[0, 2:3, :] = jnp.full((1, 128), jnp.sum(cls_t * posf),
                                    jnp.float32)
    stats_ref[0, 3:4, :] = jnp.zeros((1, 128), jnp.float32)


def _combine_body(cls_ref, stats_ref, out_ref):
    cls = cls_ref[...]
    stats = stats_ref[...]
    locsum = jnp.sum(stats[:, 0:1, 0:1])
    npos = jnp.sum(stats[:, 1:2, 0:1])
    clspos = jnp.sum(stats[:, 2:3, 0:1])

    k = jnp.minimum((3.0 * npos).astype(jnp.int32), _NP)
    bits = lax.bitcast_convert_type(cls, jnp.int32)

    def step(_, carry):
        lo, hi = carry
        mid = lo + ((hi - lo) >> 1)
        cnt = jnp.sum((bits >= mid).astype(jnp.int32), axis=2,
                      keepdims=True)
        ge = cnt >= k
        return jnp.where(ge, mid, lo), jnp.where(ge, hi, mid)

    lo0 = jnp.zeros((_BS, 1, 1), jnp.int32)
    hi0 = jnp.full((_BS, 1, 1), 0x7F800000, jnp.int32)
    lo, _ = lax.fori_loop(0, 31, step, (lo0, hi0))
    tval = lax.bitcast_convert_type(lo, jnp.float32)
    gt = bits > lo
    cnt_gt = jnp.sum(gt.astype(jnp.float32), axis=2, keepdims=True)
    sum_gt = jnp.sum(jnp.where(gt, cls, 0.0), axis=2, keepdims=True)
    topk = jnp.sum(sum_gt + (k.astype(jnp.float32) - cnt_gt) * tval)

    loss = locsum / (npos * 4.0) + (clspos + topk) / npos
    out_ref[...] = jnp.full((1, 1), loss, jnp.float32)


@jax.jit
def kernel(pred_boxes, pred_scores, true_boxes, true_classes, pboxes):
    f32 = jnp.float32
    pbc_t = pboxes.T
    pbx_t = jnp.concatenate([pbc_t[:2] - pbc_t[2:] / 2.0,
                             pbc_t[:2] + pbc_t[2:] / 2.0], axis=0)
    pad = jnp.full((4, _NPP - _NP), 2.0, f32)
    pxy = jnp.concatenate([pbx_t, pad], axis=1)
    obj4 = jnp.transpose(true_boxes, (0, 2, 1))
    tc3 = true_classes.reshape(_BS, _NO, 1).astype(jnp.int32)
    pb_t = jnp.transpose(pred_boxes, (0, 2, 1))

    sel = _sc_match(obj4, pxy)
    sel3 = sel.reshape(_BS, 1, _NPP)

    cls_all, stats = pl.pallas_call(
        _image_body,
        grid=(_BS,),
        in_specs=[
            pl.BlockSpec((1, 1, _NPP), lambda i: (i, 0, 0)),
            pl.BlockSpec((1, _NO, 4), lambda i: (i, 0, 0)),
            pl.BlockSpec((1, _NO, 1), lambda i: (i, 0, 0)),
            pl.BlockSpec((4, _NP), lambda i: (0, 0)),
            pl.BlockSpec((1, 4, _NP), lambda i: (i, 0, 0)),
            pl.BlockSpec((1, _NP, _NC), lambda i: (i, 0, 0)),
        ],
        out_specs=[
            pl.BlockSpec((1, 1, _NP), lambda i: (i, 0, 0)),
            pl.BlockSpec((1, 4, 128), lambda i: (i, 0, 0)),
        ],
        out_shape=[
            jax.ShapeDtypeStruct((_BS, 1, _NP), f32),
            jax.ShapeDtypeStruct((_BS, 4, 128), f32),
        ],
        compiler_params=pltpu.CompilerParams(
            vmem_limit_bytes=100 * 1024 * 1024),
    )(sel3, true_boxes, tc3, pbc_t, pb_t, pred_scores)

    out = pl.pallas_call(
        _combine_body,
        out_shape=jax.ShapeDtypeStruct((1, 1), f32),
    )(cls_all, stats)
    return out[0, 0]
